# Initial kernel scaffold; baseline (speedup 1.0000x reference)
import jax
import jax.numpy as jnp
from jax.experimental import pallas as pl


def kernel(x, edge_index, y, train_mask, params):
    raise NotImplementedError("write your pallas kernel here")


# speedup vs baseline: 1.1070x; 1.1070x over previous
import functools

import jax
import jax.numpy as jnp
from jax.experimental import pallas as pl

N = 10000
E = 160000
D = 256
HID = 256
VAL = 64
SEMB = 8
NC = 40
K = 3
L = 2
TAU = 0.5
LAMBD = 0.1


def _softshrink(x, l):
    return jnp.where(x > l, x - l, jnp.where(x < -l, x + l, jnp.zeros_like(x)))


def _seg_sum(vals, idx, n):
    return jax.ops.segment_sum(vals, idx, num_segments=n)


def _forward(x, edge_index, y, train_mask, p):
    n = x.shape[0]
    src = edge_index[0]
    dst = edge_index[1]

    deg = _seg_sum(jnp.ones(E, jnp.float32), dst, n) + 1.0
    dinv = 1.0 / jnp.sqrt(jnp.maximum(deg, 1.0))

    def gcn(h_in, W, b):
        g = dinv[:, None] * (h_in @ W)
        agg = _seg_sum(g[src], dst, n) + g
        return dinv[:, None] * agg + b

    h1 = jax.nn.relu(gcn(x, p['bb1_W'], p['bb1_b']))
    h2 = gcn(h1, p['bb2_W'], p['bb2_b']) + x @ p['proj_W']
    H0 = jax.nn.relu(h2)

    onehot = jax.nn.one_hot(y, NC, dtype=x.dtype)
    label_feat = onehot * train_mask.astype(x.dtype)[:, None]
    x_in = jnp.concatenate([x, label_feat], axis=-1)

    def gat(h_in, W, a_s, a_d, b):
        h = h_in @ W
        es = h @ a_s
        ed = h @ a_d
        e_edge = jax.nn.leaky_relu(es[src] + ed[dst], negative_slope=0.2)
        e_self = jax.nn.leaky_relu(es + ed, negative_slope=0.2)
        ee_edge = jnp.exp(e_edge)
        ee_self = jnp.exp(e_self)
        den = _seg_sum(ee_edge, dst, n) + ee_self + 1e-16
        alpha_e = ee_edge / den[dst]
        alpha_s = ee_self / den
        out = _seg_sum(alpha_e[:, None] * h[src], dst, n) + alpha_s[:, None] * h
        return out + b

    h = jax.nn.relu(gat(x_in, p['gat1_W'], p['gat1_as'], p['gat1_ad'], p['gat1_b']))
    h = gat(h, p['gat2_W'], p['gat2_as'], p['gat2_ad'], p['gat2_b'])

    A = h @ p['em1_W'][:HID]
    B = h @ p['em1_W'][HID:]
    hidden = jax.nn.relu(A[src] + B[dst] + p['em1_b'])
    edge_logits = hidden @ p['em2_W'] + p['em2_b']

    gkey = jax.random.key(42)
    probs_acc = jnp.zeros((n, NC), jnp.float32)
    sparse_acc = 0.0
    for k in range(K):
        g = jax.random.gumbel(jax.random.fold_in(gkey, k), (E, 3), dtype=x.dtype)
        sign_idx = jnp.argmax(edge_logits + g, axis=-1)
        edge_sign = sign_idx.astype(x.dtype) - 1.0
        H = H0
        ss = 0.0
        for l in range(L):
            pref = 'l%d_' % l
            am1W = p[pref + 'am1W']
            Wt2 = p[pref + 'Wt'] @ am1W[:VAL]
            Wv2 = p[pref + 'Wv'] @ am1W[VAL:2 * VAL]
            semb2 = p[pref + 'semb'] @ am1W[2 * VAL:]
            TT = H @ Wt2
            VV = H @ Wv2
            V = H @ p[pref + 'Wv']
            base = TT[dst] + VV[src] + p[pref + 'am1b']
            hdd = jax.nn.relu(base + semb2[sign_idx])
            alpha = (hdd @ p[pref + 'am2W'] + p[pref + 'am2b'])[:, 0]
            alpha = _softshrink(alpha, LAMBD)
            ss = ss + jnp.mean(jnp.abs(alpha))
            gamma = jax.nn.softplus(p[pref + 'gamma'])
            weff = jnp.where(edge_sign > 0, alpha,
                             jnp.where(edge_sign < 0, -gamma * jnp.abs(alpha), 0.0))
            signed = _seg_sum(weff[:, None] * V[src], dst, n)
            H = signed @ p[pref + 'WoutW'] + p[pref + 'Woutb'] + H @ p[pref + 'Wself'] + H
            H = jax.nn.relu(H)
        probs_acc = probs_acc + jax.nn.softmax(H @ p['cls_W'] + p['cls_b'], axis=-1)
        sparse_acc = sparse_acc + ss / L
    probs_mc = probs_acc / K
    logits_mc = jnp.log(probs_mc + 1e-12)
    sparse_loss = sparse_acc / K
    return logits_mc, sparse_loss


def kernel(x, edge_index, y, train_mask, params):
    return _forward(x, edge_index, y, train_mask, params)


# speedup vs baseline: 1.1144x; 1.0067x over previous
import functools

import jax
import jax.numpy as jnp
from jax import lax
from jax.experimental import pallas as pl
from jax.experimental.pallas import tpu as pltpu
from jax.experimental.pallas import tpu_sc as plsc

N = 10000
E = 160000
D = 256
HID = 256
VAL = 64
SEMB = 8
NC = 40
K = 3
L = 2
TAU = 0.5
LAMBD = 0.1


def _softshrink(x, l):
    return jnp.where(x > l, x - l, jnp.where(x < -l, x + l, jnp.zeros_like(x)))


def _seg_sum(vals, idx, n):
    return jax.ops.segment_sum(vals, idx, num_segments=n)


NPAD = 10240
EPAD = 163840
CH = 128
NW = 32
EPW = EPAD // NW
NCHUNK = EPW // CH
ROWS_PT = NPAD // 16

_MESH = plsc.VectorSubcoreMesh(core_axis_name="c", subcore_axis_name="s")


@functools.partial(jax.jit, static_argnames=("dw",))
def _sc_gather(tab, idx_pad, dw):
    @functools.partial(
        pl.kernel, mesh=_MESH,
        out_type=jax.ShapeDtypeStruct((EPAD, dw), jnp.float32),
        compiler_params=pltpu.CompilerParams(use_tc_tiling_on_sc=False),
        scratch_types=[
            pltpu.VMEM((NCHUNK, CH), jnp.int32),
            pltpu.VMEM((CH, dw), jnp.float32),
            pltpu.SemaphoreType.DMA,
        ],
    )
    def gath(tab_hbm, idx_hbm, out_hbm, idx_v, rbuf, sem):
        wid = lax.axis_index("s") * 2 + lax.axis_index("c")
        base = wid * EPW

        def body(j, carry):
            off = base + j * CH
            pltpu.sync_copy(idx_hbm.at[pl.ds(off, CH)], idx_v.at[j])
            pltpu.async_copy(tab_hbm.at[idx_v.at[j]], rbuf, sem).wait()
            pltpu.sync_copy(rbuf, out_hbm.at[pl.ds(off, CH)])
            return carry

        lax.fori_loop(0, NCHUNK, body, 0)

    return gath(tab, idx_pad)


@functools.partial(jax.jit, static_argnames=("dw",))
def _sc_scatter(vals_pad, idx_pad, dw):
    @functools.partial(
        pl.kernel, mesh=_MESH,
        out_type=jax.ShapeDtypeStruct((2, NPAD, dw), jnp.float32),
        compiler_params=pltpu.CompilerParams(use_tc_tiling_on_sc=False),
        scratch_types=[
            pltpu.VMEM((NCHUNK, CH), jnp.int32),
            pltpu.VMEM((CH, dw), jnp.float32),
            pltpu.VMEM_SHARED((NPAD, dw), jnp.float32),
        ],
    )
    def scat(vals_hbm, idx_hbm, out_hbm, idx_v, vbuf, acc):
        c = lax.axis_index("c")
        s = lax.axis_index("s")
        wid = s * 2 + c

        def zrow(r, carry):
            def zcol(jj, cc):
                vbuf[r, pl.ds(jj * 16, 16)] = jnp.zeros((16,), jnp.float32)
                return cc
            return lax.fori_loop(0, dw // 16, zcol, carry)

        lax.fori_loop(0, CH, zrow, 0)

        def zcopy(t, carry):
            pltpu.sync_copy(vbuf, acc.at[pl.ds(s * ROWS_PT + t * CH, CH)])
            return carry

        lax.fori_loop(0, ROWS_PT // CH, zcopy, 0)
        plsc.subcore_barrier()

        base = wid * EPW

        def body(j, carry):
            off = base + j * CH
            pltpu.sync_copy(idx_hbm.at[pl.ds(off, CH)], idx_v.at[j])
            pltpu.sync_copy(vals_hbm.at[pl.ds(off, CH)], vbuf)
            pltpu.sync_copy(vbuf, acc.at[idx_v.at[j]], add=True)
            return carry

        lax.fori_loop(0, NCHUNK, body, 0)
        plsc.subcore_barrier()

        def drain(t, carry):
            r0 = s * ROWS_PT + t * CH
            pltpu.sync_copy(acc.at[pl.ds(r0, CH)], vbuf)
            pltpu.sync_copy(vbuf, out_hbm.at[c, pl.ds(r0, CH)])
            return carry

        lax.fori_loop(0, ROWS_PT // CH, drain, 0)

    return scat(vals_pad, idx_pad)


def _pad_e(a):
    pad = [(0, EPAD - E)] + [(0, 0)] * (a.ndim - 1)
    return jnp.pad(a, pad)


def _gather_rows(tab, idx_pad):
    return _sc_gather(tab, idx_pad, tab.shape[-1])[:E]


def _scatter_rows(vals, idx_pad, n):
    dw = vals.shape[-1]
    vp = _pad_e(vals)
    if dw <= 128:
        ps = _sc_scatter(vp, idx_pad, dw)
        return ps[0, :n] + ps[1, :n]
    out = []
    for c0 in range(0, dw, 128):
        ps = _sc_scatter(vp[:, c0:c0 + 128], idx_pad, 128)
        out.append(ps[0, :n] + ps[1, :n])
    return jnp.concatenate(out, axis=-1)


def _forward(x, edge_index, y, train_mask, p):
    n = x.shape[0]
    src = edge_index[0]
    dst = edge_index[1]
    src_pad = jnp.pad(src, (0, EPAD - E))
    dst_pad = jnp.pad(dst, (0, EPAD - E))

    deg = _seg_sum(jnp.ones(E, jnp.float32), dst, n) + 1.0
    dinv = 1.0 / jnp.sqrt(jnp.maximum(deg, 1.0))

    def gcn(h_in, W, b):
        g = dinv[:, None] * (h_in @ W)
        agg = _scatter_rows(_gather_rows(g, src_pad), dst_pad, n) + g
        return dinv[:, None] * agg + b

    h1 = jax.nn.relu(gcn(x, p['bb1_W'], p['bb1_b']))
    h2 = gcn(h1, p['bb2_W'], p['bb2_b']) + x @ p['proj_W']
    H0 = jax.nn.relu(h2)

    onehot = jax.nn.one_hot(y, NC, dtype=x.dtype)
    label_feat = onehot * train_mask.astype(x.dtype)[:, None]
    x_in = jnp.concatenate([x, label_feat], axis=-1)

    def gat(h_in, W, a_s, a_d, b):
        h = h_in @ W
        es = h @ a_s
        ed = h @ a_d
        e_edge = jax.nn.leaky_relu(es[src] + ed[dst], negative_slope=0.2)
        e_self = jax.nn.leaky_relu(es + ed, negative_slope=0.2)
        ee_edge = jnp.exp(e_edge)
        ee_self = jnp.exp(e_self)
        den = _seg_sum(ee_edge, dst, n) + ee_self + 1e-16
        alpha_e = ee_edge / den[dst]
        alpha_s = ee_self / den
        hsrc = _gather_rows(h, src_pad)
        out = _scatter_rows(alpha_e[:, None] * hsrc, dst_pad, n) + alpha_s[:, None] * h
        return out + b

    h = jax.nn.relu(gat(x_in, p['gat1_W'], p['gat1_as'], p['gat1_ad'], p['gat1_b']))
    h = gat(h, p['gat2_W'], p['gat2_as'], p['gat2_ad'], p['gat2_b'])

    A = h @ p['em1_W'][:HID]
    B = h @ p['em1_W'][HID:]
    hidden = jax.nn.relu(_gather_rows(A, src_pad) + _gather_rows(B, dst_pad)
                         + p['em1_b'])
    edge_logits = hidden @ p['em2_W'] + p['em2_b']

    gkey = jax.random.key(42)
    probs_acc = jnp.zeros((n, NC), jnp.float32)
    sparse_acc = 0.0
    for k in range(K):
        g = jax.random.gumbel(jax.random.fold_in(gkey, k), (E, 3), dtype=x.dtype)
        sign_idx = jnp.argmax(edge_logits + g, axis=-1)
        edge_sign = sign_idx.astype(x.dtype) - 1.0
        H = H0
        ss = 0.0
        for l in range(L):
            pref = 'l%d_' % l
            am1W = p[pref + 'am1W']
            Wt2 = p[pref + 'Wt'] @ am1W[:VAL]
            Wv2 = p[pref + 'Wv'] @ am1W[VAL:2 * VAL]
            semb2 = p[pref + 'semb'] @ am1W[2 * VAL:]
            TT = H @ Wt2
            VV = H @ Wv2
            V = H @ p[pref + 'Wv']
            base = (_gather_rows(TT, dst_pad) + _gather_rows(VV, src_pad)
                    + p[pref + 'am1b'])
            hdd = jax.nn.relu(base + semb2[sign_idx])
            alpha = (hdd @ p[pref + 'am2W'] + p[pref + 'am2b'])[:, 0]
            alpha = _softshrink(alpha, LAMBD)
            ss = ss + jnp.mean(jnp.abs(alpha))
            gamma = jax.nn.softplus(p[pref + 'gamma'])
            weff = jnp.where(edge_sign > 0, alpha,
                             jnp.where(edge_sign < 0, -gamma * jnp.abs(alpha), 0.0))
            vj = _gather_rows(V, src_pad)
            signed = _scatter_rows(weff[:, None] * vj, dst_pad, n)
            H = signed @ p[pref + 'WoutW'] + p[pref + 'Woutb'] + H @ p[pref + 'Wself'] + H
            H = jax.nn.relu(H)
        probs_acc = probs_acc + jax.nn.softmax(H @ p['cls_W'] + p['cls_b'], axis=-1)
        sparse_acc = sparse_acc + ss / L
    probs_mc = probs_acc / K
    logits_mc = jnp.log(probs_mc + 1e-12)
    sparse_loss = sparse_acc / K
    return logits_mc, sparse_loss


def kernel(x, edge_index, y, train_mask, params):
    return _forward(x, edge_index, y, train_mask, params)


# speedup vs baseline: 1.1645x; 1.0450x over previous
import functools

import jax
import jax.numpy as jnp
from jax import lax
from jax.experimental import pallas as pl
from jax.experimental.pallas import tpu as pltpu
from jax.experimental.pallas import tpu_sc as plsc

N = 10000
E = 160000
D = 256
HID = 256
VAL = 64
SEMB = 8
NC = 40
K = 3
L = 2
TAU = 0.5
LAMBD = 0.1


def _softshrink(x, l):
    return jnp.where(x > l, x - l, jnp.where(x < -l, x + l, jnp.zeros_like(x)))


def _seg_sum(vals, idx, n):
    return jax.ops.segment_sum(vals, idx, num_segments=n)


NPAD = 10240
EPAD = 163840
CH = 128
NW = 32
EPW = EPAD // NW
NCHUNK = EPW // CH
ROWS_PT = NPAD // 16

_MESH = plsc.VectorSubcoreMesh(core_axis_name="c", subcore_axis_name="s")


@functools.partial(jax.jit, static_argnames=("dw",))
def _sc_gather(tab, idx2d, dw):
    @functools.partial(
        pl.kernel, mesh=_MESH,
        out_type=jax.ShapeDtypeStruct((EPAD, dw), jnp.float32),
        compiler_params=pltpu.CompilerParams(use_tc_tiling_on_sc=False),
        scratch_types=[
            pltpu.VMEM((NCHUNK, CH), jnp.int32),
            pltpu.VMEM((CH, dw), jnp.float32),
            pltpu.VMEM((CH, dw), jnp.float32),
            pltpu.SemaphoreType.DMA,
            pltpu.SemaphoreType.DMA,
            pltpu.SemaphoreType.DMA,
            pltpu.SemaphoreType.DMA,
        ],
    )
    def gath(tab_hbm, idx_hbm, out_hbm, idx_v, rb0, rb1, g0, g1, s0, s1):
        wid = lax.axis_index("s") * 2 + lax.axis_index("c")
        base = wid * EPW
        pltpu.sync_copy(idx_hbm.at[pl.ds(wid * NCHUNK, NCHUNK)], idx_v)

        def body(t, carry):
            j0 = 2 * t
            j1 = 2 * t + 1
            cg0 = pltpu.async_copy(tab_hbm.at[idx_v.at[j0]], rb0, g0)
            cg1 = pltpu.async_copy(tab_hbm.at[idx_v.at[j1]], rb1, g1)
            cg0.wait()
            cs0 = pltpu.async_copy(rb0, out_hbm.at[pl.ds(base + j0 * CH, CH)], s0)
            cg1.wait()
            cs1 = pltpu.async_copy(rb1, out_hbm.at[pl.ds(base + j1 * CH, CH)], s1)
            cs0.wait()
            cs1.wait()
            return carry

        lax.fori_loop(0, NCHUNK // 2, body, 0)

    return gath(tab, idx2d)


def _zero_acc(vbuf, acc, s, dw):
    def zrow(r, carry):
        def zcol(jj, cc):
            vbuf[r, pl.ds(jj * 16, 16)] = jnp.zeros((16,), jnp.float32)
            return cc
        return lax.fori_loop(0, dw // 16, zcol, carry)

    lax.fori_loop(0, CH, zrow, 0)

    def zcopy(t, carry):
        pltpu.sync_copy(vbuf, acc.at[pl.ds(s * ROWS_PT + t * CH, CH)])
        return carry

    lax.fori_loop(0, ROWS_PT // CH, zcopy, 0)


def _drain_acc(vbuf, acc, out_hbm, c, s):
    def drain(t, carry):
        r0 = s * ROWS_PT + t * CH
        pltpu.sync_copy(acc.at[pl.ds(r0, CH)], vbuf)
        pltpu.sync_copy(vbuf, out_hbm.at[c, pl.ds(r0, CH)])
        return carry

    lax.fori_loop(0, ROWS_PT // CH, drain, 0)


@functools.partial(jax.jit, static_argnames=("dw",))
def _sc_scatter(vals_pad, idx2d, dw):
    @functools.partial(
        pl.kernel, mesh=_MESH,
        out_type=jax.ShapeDtypeStruct((2, NPAD, dw), jnp.float32),
        compiler_params=pltpu.CompilerParams(use_tc_tiling_on_sc=False),
        scratch_types=[
            pltpu.VMEM((NCHUNK, CH), jnp.int32),
            pltpu.VMEM((CH, dw), jnp.float32),
            pltpu.VMEM((CH, dw), jnp.float32),
            pltpu.VMEM_SHARED((NPAD, dw), jnp.float32),
            pltpu.SemaphoreType.DMA,
            pltpu.SemaphoreType.DMA,
            pltpu.SemaphoreType.DMA,
            pltpu.SemaphoreType.DMA,
        ],
    )
    def scat(vals_hbm, idx_hbm, out_hbm, idx_v, vb0, vb1, acc, l0, l1, a0, a1):
        c = lax.axis_index("c")
        s = lax.axis_index("s")
        wid = s * 2 + c
        _zero_acc(vb0, acc, s, dw)
        pltpu.sync_copy(idx_hbm.at[pl.ds(wid * NCHUNK, NCHUNK)], idx_v)
        plsc.subcore_barrier()

        base = wid * EPW

        def body(t, carry):
            j0 = 2 * t
            j1 = 2 * t + 1
            cl0 = pltpu.async_copy(vals_hbm.at[pl.ds(base + j0 * CH, CH)], vb0, l0)
            cl1 = pltpu.async_copy(vals_hbm.at[pl.ds(base + j1 * CH, CH)], vb1, l1)
            cl0.wait()
            ca0 = pltpu.async_copy(vb0, acc.at[idx_v.at[j0]], a0, add=True)
            cl1.wait()
            ca1 = pltpu.async_copy(vb1, acc.at[idx_v.at[j1]], a1, add=True)
            ca0.wait()
            ca1.wait()
            return carry

        lax.fori_loop(0, NCHUNK // 2, body, 0)
        plsc.subcore_barrier()
        _drain_acc(vb0, acc, out_hbm, c, s)

    return scat(vals_pad, idx2d)


@functools.partial(jax.jit, static_argnames=("dw",))
def _sc_spmm(tab, src2d, dst2d, dw):
    @functools.partial(
        pl.kernel, mesh=_MESH,
        out_type=jax.ShapeDtypeStruct((2, NPAD, dw), jnp.float32),
        compiler_params=pltpu.CompilerParams(use_tc_tiling_on_sc=False),
        scratch_types=[
            pltpu.VMEM((NCHUNK, CH), jnp.int32),
            pltpu.VMEM((NCHUNK, CH), jnp.int32),
            pltpu.VMEM((CH, dw), jnp.float32),
            pltpu.VMEM((CH, dw), jnp.float32),
            pltpu.VMEM_SHARED((NPAD, dw), jnp.float32),
            pltpu.SemaphoreType.DMA,
            pltpu.SemaphoreType.DMA,
            pltpu.SemaphoreType.DMA,
            pltpu.SemaphoreType.DMA,
        ],
    )
    def spmm(tab_hbm, src_hbm, dst_hbm, out_hbm,
             idx_s, idx_d, vb0, vb1, acc, g0, g1, a0, a1):
        c = lax.axis_index("c")
        s = lax.axis_index("s")
        wid = s * 2 + c
        _zero_acc(vb0, acc, s, dw)
        pltpu.sync_copy(src_hbm.at[pl.ds(wid * NCHUNK, NCHUNK)], idx_s)
        pltpu.sync_copy(dst_hbm.at[pl.ds(wid * NCHUNK, NCHUNK)], idx_d)
        plsc.subcore_barrier()

        def body(t, carry):
            j0 = 2 * t
            j1 = 2 * t + 1
            cg0 = pltpu.async_copy(tab_hbm.at[idx_s.at[j0]], vb0, g0)
            cg1 = pltpu.async_copy(tab_hbm.at[idx_s.at[j1]], vb1, g1)
            cg0.wait()
            ca0 = pltpu.async_copy(vb0, acc.at[idx_d.at[j0]], a0, add=True)
            cg1.wait()
            ca1 = pltpu.async_copy(vb1, acc.at[idx_d.at[j1]], a1, add=True)
            ca0.wait()
            ca1.wait()
            return carry

        lax.fori_loop(0, NCHUNK // 2, body, 0)
        plsc.subcore_barrier()
        _drain_acc(vb0, acc, out_hbm, c, s)

    return spmm(tab, src2d, dst2d)


def _pad_e(a):
    pad = [(0, EPAD - E)] + [(0, 0)] * (a.ndim - 1)
    return jnp.pad(a, pad)


def _gather_rows(tab, idx2d):
    return _sc_gather(tab, idx2d, tab.shape[-1])[:E]


def _scatter_rows(vals, idx2d, n):
    dw = vals.shape[-1]
    vp = _pad_e(vals)
    if dw <= 128:
        ps = _sc_scatter(vp, idx2d, dw)
        return ps[0, :n] + ps[1, :n]
    out = []
    for c0 in range(0, dw, 128):
        ps = _sc_scatter(vp[:, c0:c0 + 128], idx2d, 128)
        out.append(ps[0, :n] + ps[1, :n])
    return jnp.concatenate(out, axis=-1)


def _spmm_rows(tab, src2d, dst2d, n):
    dw = tab.shape[-1]
    out = []
    for c0 in range(0, dw, 128):
        ps = _sc_spmm(tab[:, c0:c0 + 128], src2d, dst2d, 128)
        out.append(ps[0, :n] + ps[1, :n])
    return jnp.concatenate(out, axis=-1) if len(out) > 1 else out[0]


def _forward(x, edge_index, y, train_mask, p):
    n = x.shape[0]
    src = edge_index[0]
    dst = edge_index[1]
    src_pad = jnp.pad(src, (0, EPAD - E)).reshape(EPAD // CH, CH)
    dst_pad = jnp.pad(dst, (0, EPAD - E)).reshape(EPAD // CH, CH)
    dst_scat = jnp.pad(dst, (0, EPAD - E),
                       constant_values=NPAD - 1).reshape(EPAD // CH, CH)

    deg = _seg_sum(jnp.ones(E, jnp.float32), dst, n) + 1.0
    dinv = 1.0 / jnp.sqrt(jnp.maximum(deg, 1.0))

    def gcn(h_in, W, b):
        g = dinv[:, None] * (h_in @ W)
        agg = _spmm_rows(g, src_pad, dst_scat, n) + g
        return dinv[:, None] * agg + b

    h1 = jax.nn.relu(gcn(x, p['bb1_W'], p['bb1_b']))
    h2 = gcn(h1, p['bb2_W'], p['bb2_b']) + x @ p['proj_W']
    H0 = jax.nn.relu(h2)

    onehot = jax.nn.one_hot(y, NC, dtype=x.dtype)
    label_feat = onehot * train_mask.astype(x.dtype)[:, None]
    x_in = jnp.concatenate([x, label_feat], axis=-1)

    def gat(h_in, W, a_s, a_d, b):
        h = h_in @ W
        es = h @ a_s
        ed = h @ a_d
        e_edge = jax.nn.leaky_relu(es[src] + ed[dst], negative_slope=0.2)
        e_self = jax.nn.leaky_relu(es + ed, negative_slope=0.2)
        ee_edge = jnp.exp(e_edge)
        ee_self = jnp.exp(e_self)
        den = _seg_sum(ee_edge, dst, n) + ee_self + 1e-16
        alpha_e = ee_edge / den[dst]
        alpha_s = ee_self / den
        hsrc = _gather_rows(h, src_pad)
        out = _scatter_rows(alpha_e[:, None] * hsrc, dst_scat, n) + alpha_s[:, None] * h
        return out + b

    h = jax.nn.relu(gat(x_in, p['gat1_W'], p['gat1_as'], p['gat1_ad'], p['gat1_b']))
    h = gat(h, p['gat2_W'], p['gat2_as'], p['gat2_ad'], p['gat2_b'])

    A = h @ p['em1_W'][:HID]
    B = h @ p['em1_W'][HID:]
    hidden = jax.nn.relu(_gather_rows(A, src_pad) + _gather_rows(B, dst_pad)
                         + p['em1_b'])
    edge_logits = hidden @ p['em2_W'] + p['em2_b']

    gkey = jax.random.key(42)
    probs_acc = jnp.zeros((n, NC), jnp.float32)
    sparse_acc = 0.0
    for k in range(K):
        g = jax.random.gumbel(jax.random.fold_in(gkey, k), (E, 3), dtype=x.dtype)
        sign_idx = jnp.argmax(edge_logits + g, axis=-1)
        edge_sign = sign_idx.astype(x.dtype) - 1.0
        H = H0
        ss = 0.0
        for l in range(L):
            pref = 'l%d_' % l
            am1W = p[pref + 'am1W']
            Wt2 = p[pref + 'Wt'] @ am1W[:VAL]
            Wv2 = p[pref + 'Wv'] @ am1W[VAL:2 * VAL]
            semb2 = p[pref + 'semb'] @ am1W[2 * VAL:]
            TT = H @ Wt2
            VV = H @ Wv2
            V = H @ p[pref + 'Wv']
            base = (_gather_rows(TT, dst_pad) + _gather_rows(VV, src_pad)
                    + p[pref + 'am1b'])
            hdd = jax.nn.relu(base + semb2[sign_idx])
            alpha = (hdd @ p[pref + 'am2W'] + p[pref + 'am2b'])[:, 0]
            alpha = _softshrink(alpha, LAMBD)
            ss = ss + jnp.mean(jnp.abs(alpha))
            gamma = jax.nn.softplus(p[pref + 'gamma'])
            weff = jnp.where(edge_sign > 0, alpha,
                             jnp.where(edge_sign < 0, -gamma * jnp.abs(alpha), 0.0))
            vj = _gather_rows(V, src_pad)
            signed = _scatter_rows(weff[:, None] * vj, dst_scat, n)
            H = signed @ p[pref + 'WoutW'] + p[pref + 'Woutb'] + H @ p[pref + 'Wself'] + H
            H = jax.nn.relu(H)
        probs_acc = probs_acc + jax.nn.softmax(H @ p['cls_W'] + p['cls_b'], axis=-1)
        sparse_acc = sparse_acc + ss / L
    probs_mc = probs_acc / K
    logits_mc = jnp.log(probs_mc + 1e-12)
    sparse_loss = sparse_acc / K
    return logits_mc, sparse_loss


def kernel(x, edge_index, y, train_mask, params):
    return _forward(x, edge_index, y, train_mask, params)


# speedup vs baseline: 1.1969x; 1.0278x over previous
import functools

import jax
import jax.numpy as jnp
from jax import lax
from jax.experimental import pallas as pl
from jax.experimental.pallas import tpu as pltpu
from jax.experimental.pallas import tpu_sc as plsc

N = 10000
E = 160000
D = 256
HID = 256
VAL = 64
SEMB = 8
NC = 40
K = 3
L = 2
TAU = 0.5
LAMBD = 0.1


def _softshrink(x, l):
    return jnp.where(x > l, x - l, jnp.where(x < -l, x + l, jnp.zeros_like(x)))


def _seg_sum(vals, idx, n):
    return jax.ops.segment_sum(vals, idx, num_segments=n)


NPAD = 10240
EPAD = 163840
CH = 128
NW = 32
EPW = EPAD // NW
NCHUNK = EPW // CH
ROWS_PT = NPAD // 16

_MESH = plsc.VectorSubcoreMesh(core_axis_name="c", subcore_axis_name="s")


@functools.partial(jax.jit, static_argnames=("dw", "dt"))
def _sc_gather(tab, idx2d, dw, dt):
    @functools.partial(
        pl.kernel, mesh=_MESH,
        out_type=jax.ShapeDtypeStruct((EPAD, dw), dt),
        compiler_params=pltpu.CompilerParams(use_tc_tiling_on_sc=False),
        scratch_types=[
            pltpu.VMEM((NCHUNK, CH), jnp.int32),
            pltpu.VMEM((CH, dw), dt),
            pltpu.VMEM((CH, dw), dt),
            pltpu.SemaphoreType.DMA,
            pltpu.SemaphoreType.DMA,
            pltpu.SemaphoreType.DMA,
            pltpu.SemaphoreType.DMA,
        ],
    )
    def gath(tab_hbm, idx_hbm, out_hbm, idx_v, rb0, rb1, g0, g1, s0, s1):
        wid = lax.axis_index("s") * 2 + lax.axis_index("c")
        base = wid * EPW
        pltpu.sync_copy(idx_hbm.at[pl.ds(wid * NCHUNK, NCHUNK)], idx_v)

        def body(t, carry):
            j0 = 2 * t
            j1 = 2 * t + 1
            cg0 = pltpu.async_copy(tab_hbm.at[idx_v.at[j0]], rb0, g0)
            cg1 = pltpu.async_copy(tab_hbm.at[idx_v.at[j1]], rb1, g1)
            cg0.wait()
            cs0 = pltpu.async_copy(rb0, out_hbm.at[pl.ds(base + j0 * CH, CH)], s0)
            cg1.wait()
            cs1 = pltpu.async_copy(rb1, out_hbm.at[pl.ds(base + j1 * CH, CH)], s1)
            cs0.wait()
            cs1.wait()
            return carry

        lax.fori_loop(0, NCHUNK // 2, body, 0)

    return gath(tab, idx2d)


def _zero_acc(vbuf, acc, s, dw):
    def zrow(r, carry):
        def zcol(jj, cc):
            vbuf[r, pl.ds(jj * 16, 16)] = jnp.zeros((16,), jnp.float32)
            return cc
        return lax.fori_loop(0, dw // 16, zcol, carry)

    lax.fori_loop(0, CH, zrow, 0)

    def zcopy(t, carry):
        pltpu.sync_copy(vbuf, acc.at[pl.ds(s * ROWS_PT + t * CH, CH)])
        return carry

    lax.fori_loop(0, ROWS_PT // CH, zcopy, 0)


def _drain_acc(vbuf, acc, out_hbm, c, s):
    def drain(t, carry):
        r0 = s * ROWS_PT + t * CH
        pltpu.sync_copy(acc.at[pl.ds(r0, CH)], vbuf)
        pltpu.sync_copy(vbuf, out_hbm.at[c, pl.ds(r0, CH)])
        return carry

    lax.fori_loop(0, ROWS_PT // CH, drain, 0)


@functools.partial(jax.jit, static_argnames=("dw",))
def _sc_scatter(vals_pad, idx2d, dw):
    @functools.partial(
        pl.kernel, mesh=_MESH,
        out_type=jax.ShapeDtypeStruct((2, NPAD, dw), jnp.float32),
        compiler_params=pltpu.CompilerParams(use_tc_tiling_on_sc=False),
        scratch_types=[
            pltpu.VMEM((NCHUNK, CH), jnp.int32),
            pltpu.VMEM((CH, dw), jnp.float32),
            pltpu.VMEM((CH, dw), jnp.float32),
            pltpu.VMEM_SHARED((NPAD, dw), jnp.float32),
            pltpu.SemaphoreType.DMA,
            pltpu.SemaphoreType.DMA,
            pltpu.SemaphoreType.DMA,
            pltpu.SemaphoreType.DMA,
        ],
    )
    def scat(vals_hbm, idx_hbm, out_hbm, idx_v, vb0, vb1, acc, l0, l1, a0, a1):
        c = lax.axis_index("c")
        s = lax.axis_index("s")
        wid = s * 2 + c
        _zero_acc(vb0, acc, s, dw)
        pltpu.sync_copy(idx_hbm.at[pl.ds(wid * NCHUNK, NCHUNK)], idx_v)
        plsc.subcore_barrier()

        base = wid * EPW

        def body(t, carry):
            j0 = 2 * t
            j1 = 2 * t + 1
            cl0 = pltpu.async_copy(vals_hbm.at[pl.ds(base + j0 * CH, CH)], vb0, l0)
            cl1 = pltpu.async_copy(vals_hbm.at[pl.ds(base + j1 * CH, CH)], vb1, l1)
            cl0.wait()
            ca0 = pltpu.async_copy(vb0, acc.at[idx_v.at[j0]], a0, add=True)
            cl1.wait()
            ca1 = pltpu.async_copy(vb1, acc.at[idx_v.at[j1]], a1, add=True)
            ca0.wait()
            ca1.wait()
            return carry

        lax.fori_loop(0, NCHUNK // 2, body, 0)
        plsc.subcore_barrier()
        _drain_acc(vb0, acc, out_hbm, c, s)

    return scat(vals_pad, idx2d)


@functools.partial(jax.jit, static_argnames=("dw",))
def _sc_spmm(tab, src2d, dst2d, dw):
    @functools.partial(
        pl.kernel, mesh=_MESH,
        out_type=jax.ShapeDtypeStruct((2, NPAD, dw), jnp.float32),
        compiler_params=pltpu.CompilerParams(use_tc_tiling_on_sc=False),
        scratch_types=[
            pltpu.VMEM((NCHUNK, CH), jnp.int32),
            pltpu.VMEM((NCHUNK, CH), jnp.int32),
            pltpu.VMEM((CH, dw), jnp.float32),
            pltpu.VMEM((CH, dw), jnp.float32),
            pltpu.VMEM_SHARED((NPAD, dw), jnp.float32),
            pltpu.SemaphoreType.DMA,
            pltpu.SemaphoreType.DMA,
            pltpu.SemaphoreType.DMA,
            pltpu.SemaphoreType.DMA,
        ],
    )
    def spmm(tab_hbm, src_hbm, dst_hbm, out_hbm,
             idx_s, idx_d, vb0, vb1, acc, g0, g1, a0, a1):
        c = lax.axis_index("c")
        s = lax.axis_index("s")
        wid = s * 2 + c
        _zero_acc(vb0, acc, s, dw)
        pltpu.sync_copy(src_hbm.at[pl.ds(wid * NCHUNK, NCHUNK)], idx_s)
        pltpu.sync_copy(dst_hbm.at[pl.ds(wid * NCHUNK, NCHUNK)], idx_d)
        plsc.subcore_barrier()

        def body(t, carry):
            j0 = 2 * t
            j1 = 2 * t + 1
            cg0 = pltpu.async_copy(tab_hbm.at[idx_s.at[j0]], vb0, g0)
            cg1 = pltpu.async_copy(tab_hbm.at[idx_s.at[j1]], vb1, g1)
            cg0.wait()
            ca0 = pltpu.async_copy(vb0, acc.at[idx_d.at[j0]], a0, add=True)
            cg1.wait()
            ca1 = pltpu.async_copy(vb1, acc.at[idx_d.at[j1]], a1, add=True)
            ca0.wait()
            ca1.wait()
            return carry

        lax.fori_loop(0, NCHUNK // 2, body, 0)
        plsc.subcore_barrier()
        _drain_acc(vb0, acc, out_hbm, c, s)

    return spmm(tab, src2d, dst2d)


def _pad_e(a):
    pad = [(0, EPAD - E)] + [(0, 0)] * (a.ndim - 1)
    return jnp.pad(a, pad)


def _gather_rows(tab, idx2d):
    return _sc_gather(tab, idx2d, tab.shape[-1], tab.dtype)[:E]


def _scatter_rows(vals, idx2d, n):
    dw = vals.shape[-1]
    vp = _pad_e(vals)
    if dw <= 128:
        ps = _sc_scatter(vp, idx2d, dw)
        return ps[0, :n] + ps[1, :n]
    out = []
    for c0 in range(0, dw, 128):
        ps = _sc_scatter(vp[:, c0:c0 + 128], idx2d, 128)
        out.append(ps[0, :n] + ps[1, :n])
    return jnp.concatenate(out, axis=-1)


def _spmm_rows(tab, src2d, dst2d, n):
    dw = tab.shape[-1]
    out = []
    for c0 in range(0, dw, 128):
        ps = _sc_spmm(tab[:, c0:c0 + 128], src2d, dst2d, 128)
        out.append(ps[0, :n] + ps[1, :n])
    return jnp.concatenate(out, axis=-1) if len(out) > 1 else out[0]


def _forward(x, edge_index, y, train_mask, p):
    n = x.shape[0]
    src = edge_index[0]
    dst = edge_index[1]
    src_pad = jnp.pad(src, (0, EPAD - E)).reshape(EPAD // CH, CH)
    dst_pad = jnp.pad(dst, (0, EPAD - E)).reshape(EPAD // CH, CH)
    dst_scat = jnp.pad(dst, (0, EPAD - E),
                       constant_values=NPAD - 1).reshape(EPAD // CH, CH)

    deg = _seg_sum(jnp.ones(E, jnp.float32), dst, n) + 1.0
    dinv = 1.0 / jnp.sqrt(jnp.maximum(deg, 1.0))

    def gcn(h_in, W, b):
        g = dinv[:, None] * (h_in @ W)
        agg = _spmm_rows(g, src_pad, dst_scat, n) + g
        return dinv[:, None] * agg + b

    h1 = jax.nn.relu(gcn(x, p['bb1_W'], p['bb1_b']))
    h2 = gcn(h1, p['bb2_W'], p['bb2_b']) + x @ p['proj_W']
    H0 = jax.nn.relu(h2)

    onehot = jax.nn.one_hot(y, NC, dtype=x.dtype)
    label_feat = onehot * train_mask.astype(x.dtype)[:, None]
    x_in = jnp.concatenate([x, label_feat], axis=-1)

    def gat(h_in, W, a_s, a_d, b):
        h = h_in @ W
        es = h @ a_s
        ed = h @ a_d
        e_edge = jax.nn.leaky_relu(es[src] + ed[dst], negative_slope=0.2)
        e_self = jax.nn.leaky_relu(es + ed, negative_slope=0.2)
        ee_edge = jnp.exp(e_edge)
        ee_self = jnp.exp(e_self)
        den = _seg_sum(ee_edge, dst, n) + ee_self + 1e-16
        alpha_e = ee_edge / den[dst]
        alpha_s = ee_self / den
        hsrc = _gather_rows(h.astype(jnp.bfloat16), src_pad).astype(jnp.float32)
        out = _scatter_rows(alpha_e[:, None] * hsrc, dst_scat, n) + alpha_s[:, None] * h
        return out + b

    h = jax.nn.relu(gat(x_in, p['gat1_W'], p['gat1_as'], p['gat1_ad'], p['gat1_b']))
    h = gat(h, p['gat2_W'], p['gat2_as'], p['gat2_ad'], p['gat2_b'])

    A = h @ p['em1_W'][:HID]
    B = h @ p['em1_W'][HID:]
    hidden = jax.nn.relu(_gather_rows(A.astype(jnp.bfloat16), src_pad).astype(jnp.float32)
                         + _gather_rows(B.astype(jnp.bfloat16), dst_pad).astype(jnp.float32)
                         + p['em1_b'])
    edge_logits = hidden @ p['em2_W'] + p['em2_b']

    gkey = jax.random.key(42)
    probs_acc = jnp.zeros((n, NC), jnp.float32)
    sparse_acc = 0.0
    for k in range(K):
        g = jax.random.gumbel(jax.random.fold_in(gkey, k), (E, 3), dtype=x.dtype)
        sign_idx = jnp.argmax(edge_logits + g, axis=-1)
        edge_sign = sign_idx.astype(x.dtype) - 1.0
        H = H0
        ss = 0.0
        for l in range(L):
            pref = 'l%d_' % l
            am1W = p[pref + 'am1W']
            Wt2 = p[pref + 'Wt'] @ am1W[:VAL]
            Wv2 = p[pref + 'Wv'] @ am1W[VAL:2 * VAL]
            semb2 = p[pref + 'semb'] @ am1W[2 * VAL:]
            TT = H @ Wt2
            VV = H @ Wv2
            V = H @ p[pref + 'Wv']
            base = (_gather_rows(TT.astype(jnp.bfloat16), dst_pad).astype(jnp.float32)
                    + _gather_rows(VV.astype(jnp.bfloat16), src_pad).astype(jnp.float32)
                    + p[pref + 'am1b'])
            hdd = jax.nn.relu(base + semb2[sign_idx])
            alpha = (hdd @ p[pref + 'am2W'] + p[pref + 'am2b'])[:, 0]
            alpha = _softshrink(alpha, LAMBD)
            ss = ss + jnp.mean(jnp.abs(alpha))
            gamma = jax.nn.softplus(p[pref + 'gamma'])
            weff = jnp.where(edge_sign > 0, alpha,
                             jnp.where(edge_sign < 0, -gamma * jnp.abs(alpha), 0.0))
            vj = _gather_rows(V.astype(jnp.bfloat16), src_pad).astype(jnp.float32)
            signed = _scatter_rows(weff[:, None] * vj, dst_scat, n)
            H = signed @ p[pref + 'WoutW'] + p[pref + 'Woutb'] + H @ p[pref + 'Wself'] + H
            H = jax.nn.relu(H)
        probs_acc = probs_acc + jax.nn.softmax(H @ p['cls_W'] + p['cls_b'], axis=-1)
        sparse_acc = sparse_acc + ss / L
    probs_mc = probs_acc / K
    logits_mc = jnp.log(probs_mc + 1e-12)
    sparse_loss = sparse_acc / K
    return logits_mc, sparse_loss


def kernel(x, edge_index, y, train_mask, params):
    return _forward(x, edge_index, y, train_mask, params)
